# baseline (speedup 1.0000x reference)
import functools

import jax
import jax.numpy as jnp
from jax import lax
from jax.experimental import pallas as pl
from jax.experimental.pallas import tpu as pltpu
from jax.experimental.pallas import tpu_sc as plsc

N = 10000
E = 320000
D = 128
DH = D // 2
NC = 2
NS = 16
NW = NC * NS
EPT = E // NW
C = 100
NCH = EPT // C
GRP = 10
NP = 10240
RPT = NP // NS
RC = 128
NRC = RPT // RC
BN_EPS = 1e-4

_mesh = plsc.VectorSubcoreMesh(core_axis_name="c", subcore_axis_name="s")
_sc_params = pltpu.CompilerParams(use_tc_tiling_on_sc=False)


@functools.partial(
    pl.kernel,
    out_type=jax.ShapeDtypeStruct((NC, 2, NP, 16), jnp.float32),
    mesh=_mesh,
    compiler_params=_sc_params,
    scratch_types=[
        pltpu.VMEM((NCH, C), jnp.int32),
        pltpu.VMEM((C, 16), jnp.float32),
        pltpu.VMEM((RPT, 16), jnp.float32),
        pltpu.VMEM_SHARED((NP, 16), jnp.float32),
        pltpu.VMEM_SHARED((NP, 16), jnp.float32),
        pltpu.SemaphoreType.DMA,
    ],
)
def _sc_degrees(src_hbm, dst_hbm, deg_hbm, idx_v, ones_v, buf_v, degin_sh,
                degout_sh, sem):
    c = lax.axis_index("c")
    s = lax.axis_index("s")
    wid = c * NS + s

    def fill(i, carry):
        buf_v[i, :] = jnp.zeros((16,), jnp.float32)
        return carry

    lax.fori_loop(0, RPT, fill, 0)

    def fill_ones(i, carry):
        ones_v[i, :] = jnp.ones((16,), jnp.float32)
        return carry

    lax.fori_loop(0, C, fill_ones, 0)
    pltpu.sync_copy(buf_v, degin_sh.at[pl.ds(s * RPT, RPT)])
    pltpu.sync_copy(buf_v, degout_sh.at[pl.ds(s * RPT, RPT)])
    plsc.subcore_barrier()

    for idx_hbm, deg_sh in ((dst_hbm, degin_sh), (src_hbm, degout_sh)):
        pltpu.sync_copy(idx_hbm.at[wid], idx_v)

        def group(g, carry):
            for k in range(GRP):
                pltpu.async_copy(ones_v, deg_sh.at[idx_v.at[g * GRP + k]],
                                 sem, add=True)
            for k in range(GRP):
                pltpu.make_async_copy(ones_v, deg_sh.at[idx_v.at[0]],
                                      sem).wait()
            return carry

        lax.fori_loop(0, NCH // GRP, group, 0)

    plsc.subcore_barrier()
    pltpu.sync_copy(degin_sh.at[pl.ds(s * RPT, RPT)], buf_v)
    pltpu.sync_copy(buf_v, deg_hbm.at[c, 0, pl.ds(s * RPT, RPT)])
    pltpu.sync_copy(degout_sh.at[pl.ds(s * RPT, RPT)], buf_v)
    pltpu.sync_copy(buf_v, deg_hbm.at[c, 1, pl.ds(s * RPT, RPT)])


@functools.partial(
    pl.kernel,
    out_type=jax.ShapeDtypeStruct((2, NC, NP, DH), jnp.float32),
    mesh=_mesh,
    compiler_params=_sc_params,
    scratch_types=[
        pltpu.VMEM((NCH, C), jnp.int32),
        pltpu.VMEM((NCH, C), jnp.int32),
        pltpu.VMEM((C, DH), jnp.float32),
        pltpu.VMEM((C, DH), jnp.float32),
        pltpu.VMEM((RC, DH), jnp.float32),
        pltpu.VMEM_SHARED((NP, DH), jnp.float32),
        pltpu.SemaphoreType.DMA,
    ],
)
def _sc_aggregate(xs_a, xs_b, src_hbm, dst_hbm, agg_hbm, srcs_v, dsts_v,
                  rows0_v, rows1_v, buf_v, agg_sh, sem):
    c = lax.axis_index("c")
    s = lax.axis_index("s")
    wid = c * NS + s
    pltpu.sync_copy(src_hbm.at[wid], srcs_v)
    pltpu.sync_copy(dst_hbm.at[wid], dsts_v)

    def zb(i, carry):
        for j in range(DH // 16):
            buf_v[i, pl.ds(j * 16, 16)] = jnp.zeros((16,), jnp.float32)
        return carry

    for half, xs_hbm in ((0, xs_a), (1, xs_b)):
        lax.fori_loop(0, RC, zb, 0)
        for k in range(NRC):
            pltpu.sync_copy(buf_v, agg_sh.at[pl.ds(s * RPT + k * RC, RC)])
        plsc.subcore_barrier()

        pltpu.async_copy(xs_hbm.at[srcs_v.at[0]], rows0_v, sem)

        def pair(h, carry):
            ch0 = 2 * h
            ch1 = ch0 + 1
            pltpu.async_copy(xs_hbm.at[srcs_v.at[ch1]], rows1_v, sem)
            pltpu.make_async_copy(xs_hbm.at[srcs_v.at[ch0]], rows0_v,
                                  sem).wait()
            pltpu.sync_copy(rows0_v, agg_sh.at[dsts_v.at[ch0]], add=True)

            @pl.when(ch1 + 1 < NCH)
            def _():
                pltpu.async_copy(xs_hbm.at[srcs_v.at[ch1 + 1]], rows0_v, sem)

            pltpu.make_async_copy(xs_hbm.at[srcs_v.at[ch1]], rows1_v,
                                  sem).wait()
            pltpu.sync_copy(rows1_v, agg_sh.at[dsts_v.at[ch1]], add=True)
            return carry

        lax.fori_loop(0, NCH // 2, pair, 0)
        plsc.subcore_barrier()
        for k in range(NRC):
            r0 = s * RPT + k * RC
            pltpu.sync_copy(agg_sh.at[pl.ds(r0, RC)], buf_v)
            pltpu.sync_copy(buf_v, agg_hbm.at[half, c, pl.ds(r0, RC)])
        plsc.subcore_barrier()


def _tc_prescale_body(x_ref, degout_ref, xs_a_ref, xs_b_ref):
    deg_out = degout_ref[0] + degout_ref[1]
    r_out = lax.rsqrt(jnp.maximum(deg_out, 1.0))
    xs = x_ref[...] * r_out
    xs_a_ref[...] = xs[:, :DH]
    xs_b_ref[...] = xs[:, DH:]


RB = 1000
NB = N // RB


def _tc_dense_body(agg_ref, degin_ref, x_ref, wg_ref, gb_ref, bb_ref, wf_ref,
                   bf_ref, o_ref, y_sc, sums_sc):
    p = pl.program_id(0)
    i = pl.program_id(1)

    @pl.when(p == 0)
    def _():
        deg_in = degin_ref[0] + degin_ref[1]
        r_in = lax.rsqrt(jnp.maximum(deg_in, 1.0))
        agg_lo = agg_ref[0, 0] + agg_ref[0, 1]
        agg_hi = agg_ref[1, 0] + agg_ref[1, 1]
        agg = jnp.concatenate([agg_lo, agg_hi], axis=1) * r_in
        y = jnp.dot(agg, wg_ref[...], preferred_element_type=jnp.float32,
                    precision=lax.Precision.HIGHEST)
        y_sc[pl.ds(i * RB, RB), :] = y

        @pl.when(i == 0)
        def _():
            sums_sc[...] = jnp.zeros_like(sums_sc)

        sums_sc[0:1, :] += jnp.sum(y, axis=0, keepdims=True)
        sums_sc[1:2, :] += jnp.sum(y * y, axis=0, keepdims=True)

    @pl.when(p == 1)
    def _():
        y = y_sc[pl.ds(i * RB, RB), :]
        mean = sums_sc[0:1, :] * (1.0 / N)
        var = sums_sc[1:2, :] * (1.0 / N) - mean * mean
        yn = gb_ref[...] * (y - mean) * lax.rsqrt(var + BN_EPS) + bb_ref[...]
        film = jnp.dot(yn, wf_ref[...], preferred_element_type=jnp.float32,
                       precision=lax.Precision.HIGHEST) + bf_ref[...]
        z = film[:, :D] * yn + film[:, D:]
        o_ref[...] = jnp.maximum(z, 0.0) + x_ref[...]


def kernel(x, edge_index, W_gcn, gamma_bn, beta_bn, W_film, b_film):
    src = edge_index[0].astype(jnp.int32).reshape(NW, NCH, C)
    dst = edge_index[1].astype(jnp.int32).reshape(NW, NCH, C)
    deg = _sc_degrees(src, dst)
    degin = deg[:, 0, :N, 0:1]
    degout = deg[:, 1, :N, 0:1]
    xs_a, xs_b = pl.pallas_call(
        _tc_prescale_body,
        out_shape=(jax.ShapeDtypeStruct((N, DH), jnp.float32),
                   jax.ShapeDtypeStruct((N, DH), jnp.float32)),
    )(x, degout)
    agg = _sc_aggregate(xs_a, xs_b, src, dst)
    out = pl.pallas_call(
        _tc_dense_body,
        grid=(2, NB),
        in_specs=[
            pl.BlockSpec((2, NC, RB, DH), lambda p, i: (0, 0, i * (1 - p), 0)),
            pl.BlockSpec((NC, RB, 1), lambda p, i: (0, i * (1 - p), 0)),
            pl.BlockSpec((RB, D), lambda p, i: (i * p, 0)),
            pl.BlockSpec((D, D), lambda p, i: (0, 0)),
            pl.BlockSpec((1, D), lambda p, i: (0, 0)),
            pl.BlockSpec((1, D), lambda p, i: (0, 0)),
            pl.BlockSpec((D, 2 * D), lambda p, i: (0, 0)),
            pl.BlockSpec((1, 2 * D), lambda p, i: (0, 0)),
        ],
        out_specs=pl.BlockSpec((RB, D), lambda p, i: (i * p, 0)),
        out_shape=jax.ShapeDtypeStruct((N, D), jnp.float32),
        scratch_shapes=[
            pltpu.VMEM((N, D), jnp.float32),
            pltpu.VMEM((8, D), jnp.float32),
        ],
    )(agg, degin, x, W_gcn, gamma_bn.reshape(1, D), beta_bn.reshape(1, D),
      W_film, b_film.reshape(1, 2 * D))
    return out


# speedup vs baseline: 1.0490x; 1.0490x over previous
import functools

import jax
import jax.numpy as jnp
from jax import lax
from jax.experimental import pallas as pl
from jax.experimental.pallas import tpu as pltpu
from jax.experimental.pallas import tpu_sc as plsc

N = 10000
E = 320000
D = 128
DH = D // 2
NC = 2
NS = 16
NW = NC * NS
EPT = E // NW
C = 80
NCH = EPT // C
GRP = 5
NP = 10240
RPT = NP // NS
RC = 128
NRC = RPT // RC
BN_EPS = 1e-4

_mesh = plsc.VectorSubcoreMesh(core_axis_name="c", subcore_axis_name="s")
_sc_params = pltpu.CompilerParams(use_tc_tiling_on_sc=False,
                                  needs_layout_passes=False)


@functools.partial(
    pl.kernel,
    out_type=jax.ShapeDtypeStruct((NC, 2, NP), jnp.float32),
    mesh=_mesh,
    compiler_params=_sc_params,
    scratch_types=[
        pltpu.VMEM((EPT,), jnp.int32),
        pltpu.VMEM((C, 16), jnp.float32),
        pltpu.VMEM((RPT, 16), jnp.float32),
        pltpu.VMEM((RPT,), jnp.float32),
        pltpu.VMEM_SHARED((NP, 16), jnp.float32),
        pltpu.VMEM_SHARED((NP, 16), jnp.float32),
        pltpu.SemaphoreType.DMA,
    ],
)
def _sc_degrees(ei_hbm, deg_hbm, idx_v, ones_v, buf_v, cbuf_v, degin_sh,
                degout_sh, sem):
    c = lax.axis_index("c")
    s = lax.axis_index("s")
    base = (c * NS + s) * EPT

    def fill(i, carry):
        buf_v[i, :] = jnp.zeros((16,), jnp.float32)
        return carry

    lax.fori_loop(0, RPT, fill, 0)

    def fill_ones(i, carry):
        ones_v[i, :] = jnp.ones((16,), jnp.float32)
        return carry

    lax.fori_loop(0, C, fill_ones, 0)
    pltpu.sync_copy(buf_v, degin_sh.at[pl.ds(s * RPT, RPT)])
    pltpu.sync_copy(buf_v, degout_sh.at[pl.ds(s * RPT, RPT)])
    plsc.subcore_barrier()

    for row, deg_sh in ((1, degin_sh), (0, degout_sh)):
        pltpu.sync_copy(ei_hbm.at[row, pl.ds(base, EPT)], idx_v)

        def group(g, carry):
            for k in range(GRP):
                pltpu.async_copy(
                    ones_v, deg_sh.at[idx_v.at[pl.ds((g * GRP + k) * C, C)]],
                    sem, add=True)
            for k in range(GRP):
                pltpu.make_async_copy(ones_v,
                                      deg_sh.at[idx_v.at[pl.ds(0, C)]],
                                      sem).wait()
            return carry

        lax.fori_loop(0, NCH // GRP, group, 0)

    plsc.subcore_barrier()
    lane0 = jnp.zeros((16,), jnp.int32)
    for t, deg_sh in ((0, degin_sh), (1, degout_sh)):
        pltpu.sync_copy(deg_sh.at[pl.ds(s * RPT, RPT)], buf_v)

        def compact(r, carry):
            rows16 = lax.iota(jnp.int32, 16) + r * 16
            cbuf_v[pl.ds(r * 16, 16)] = plsc.load_gather(
                buf_v, [rows16, lane0])
            return carry

        lax.fori_loop(0, RPT // 16, compact, 0)
        pltpu.sync_copy(cbuf_v, deg_hbm.at[c, t, pl.ds(s * RPT, RPT)])


@functools.partial(
    pl.kernel,
    out_type=jax.ShapeDtypeStruct((2, NC, NP, DH), jnp.float32),
    mesh=_mesh,
    compiler_params=_sc_params,
    scratch_types=[
        pltpu.VMEM((EPT,), jnp.int32),
        pltpu.VMEM((EPT,), jnp.int32),
        pltpu.VMEM((C, DH), jnp.float32),
        pltpu.VMEM((C, DH), jnp.float32),
        pltpu.VMEM((RC, DH), jnp.float32),
        pltpu.VMEM_SHARED((NP, DH), jnp.float32),
        pltpu.SemaphoreType.DMA,
    ],
)
def _sc_aggregate(xs_a, xs_b, ei_hbm, agg_hbm, srcs_v, dsts_v,
                  rows0_v, rows1_v, buf_v, agg_sh, sem):
    c = lax.axis_index("c")
    s = lax.axis_index("s")
    base = (c * NS + s) * EPT
    pltpu.sync_copy(ei_hbm.at[0, pl.ds(base, EPT)], srcs_v)
    pltpu.sync_copy(ei_hbm.at[1, pl.ds(base, EPT)], dsts_v)

    def zb(i, carry):
        for j in range(DH // 16):
            buf_v[i, pl.ds(j * 16, 16)] = jnp.zeros((16,), jnp.float32)
        return carry

    for half, xs_hbm in ((0, xs_a), (1, xs_b)):
        lax.fori_loop(0, RC, zb, 0)
        for k in range(NRC):
            pltpu.sync_copy(buf_v, agg_sh.at[pl.ds(s * RPT + k * RC, RC)])
        plsc.subcore_barrier()

        def src_at(ch):
            return srcs_v.at[pl.ds(ch * C, C)]

        def dst_at(ch):
            return dsts_v.at[pl.ds(ch * C, C)]

        pltpu.async_copy(xs_hbm.at[src_at(0)], rows0_v, sem)

        def pair(h, carry):
            ch0 = 2 * h
            ch1 = ch0 + 1
            pltpu.async_copy(xs_hbm.at[src_at(ch1)], rows1_v, sem)
            pltpu.make_async_copy(xs_hbm.at[src_at(ch0)], rows0_v,
                                  sem).wait()
            pltpu.sync_copy(rows0_v, agg_sh.at[dst_at(ch0)], add=True)

            @pl.when(ch1 + 1 < NCH)
            def _():
                pltpu.async_copy(xs_hbm.at[src_at(ch1 + 1)], rows0_v, sem)

            pltpu.make_async_copy(xs_hbm.at[src_at(ch1)], rows1_v,
                                  sem).wait()
            pltpu.sync_copy(rows1_v, agg_sh.at[dst_at(ch1)], add=True)
            return carry

        lax.fori_loop(0, NCH // 2, pair, 0)
        if NCH % 2:
            pltpu.make_async_copy(xs_hbm.at[src_at(NCH - 1)], rows0_v,
                                  sem).wait()
            pltpu.sync_copy(rows0_v, agg_sh.at[dst_at(NCH - 1)], add=True)
        plsc.subcore_barrier()
        for k in range(NRC):
            r0 = s * RPT + k * RC
            pltpu.sync_copy(agg_sh.at[pl.ds(r0, RC)], buf_v)
            pltpu.sync_copy(buf_v, agg_hbm.at[half, c, pl.ds(r0, RC)])
        plsc.subcore_barrier()


def _tc_prescale_body(x_ref, degout_ref, xs_a_ref, xs_b_ref):
    deg_out = degout_ref[0] + degout_ref[1]
    r_out = lax.rsqrt(jnp.maximum(deg_out, 1.0))
    xs = x_ref[...] * r_out
    xs_a_ref[...] = xs[:, :DH]
    xs_b_ref[...] = xs[:, DH:]


RB = 1000
NB = N // RB


def _tc_dense_body(agg_ref, degin_ref, x_ref, wg_ref, gb_ref, bb_ref, wf_ref,
                   bf_ref, o_ref, y_sc, sums_sc):
    p = pl.program_id(0)
    i = pl.program_id(1)

    @pl.when(p == 0)
    def _():
        deg_in = degin_ref[0] + degin_ref[1]
        r_in = lax.rsqrt(jnp.maximum(deg_in, 1.0))
        agg_lo = agg_ref[0, 0] + agg_ref[0, 1]
        agg_hi = agg_ref[1, 0] + agg_ref[1, 1]
        agg = jnp.concatenate([agg_lo, agg_hi], axis=1) * r_in
        y = jnp.dot(agg, wg_ref[...], preferred_element_type=jnp.float32,
                    precision=lax.Precision.HIGHEST)
        y_sc[pl.ds(i * RB, RB), :] = y

        @pl.when(i == 0)
        def _():
            sums_sc[...] = jnp.zeros_like(sums_sc)

        sums_sc[0:1, :] += jnp.sum(y, axis=0, keepdims=True)
        sums_sc[1:2, :] += jnp.sum(y * y, axis=0, keepdims=True)

    @pl.when(p == 1)
    def _():
        y = y_sc[pl.ds(i * RB, RB), :]
        mean = sums_sc[0:1, :] * (1.0 / N)
        var = sums_sc[1:2, :] * (1.0 / N) - mean * mean
        yn = gb_ref[...] * (y - mean) * lax.rsqrt(var + BN_EPS) + bb_ref[...]
        film = jnp.dot(yn, wf_ref[...], preferred_element_type=jnp.float32,
                       precision=lax.Precision.HIGHEST) + bf_ref[...]
        z = film[:, :D] * yn + film[:, D:]
        o_ref[...] = jnp.maximum(z, 0.0) + x_ref[...]


def kernel(x, edge_index, W_gcn, gamma_bn, beta_bn, W_film, b_film):
    ei = edge_index.astype(jnp.int32)
    deg = _sc_degrees(ei)
    degin = deg[:, 0, :N, None]
    degout = deg[:, 1, :N, None]
    xs_a, xs_b = pl.pallas_call(
        _tc_prescale_body,
        out_shape=(jax.ShapeDtypeStruct((N, DH), jnp.float32),
                   jax.ShapeDtypeStruct((N, DH), jnp.float32)),
    )(x, degout)
    agg = _sc_aggregate(xs_a, xs_b, ei)
    out = pl.pallas_call(
        _tc_dense_body,
        grid=(2, NB),
        in_specs=[
            pl.BlockSpec((2, NC, RB, DH), lambda p, i: (0, 0, i * (1 - p), 0)),
            pl.BlockSpec((NC, RB, 1), lambda p, i: (0, i * (1 - p), 0)),
            pl.BlockSpec((RB, D), lambda p, i: (i * p, 0)),
            pl.BlockSpec((D, D), lambda p, i: (0, 0)),
            pl.BlockSpec((1, D), lambda p, i: (0, 0)),
            pl.BlockSpec((1, D), lambda p, i: (0, 0)),
            pl.BlockSpec((D, 2 * D), lambda p, i: (0, 0)),
            pl.BlockSpec((1, 2 * D), lambda p, i: (0, 0)),
        ],
        out_specs=pl.BlockSpec((RB, D), lambda p, i: (i * p, 0)),
        out_shape=jax.ShapeDtypeStruct((N, D), jnp.float32),
        scratch_shapes=[
            pltpu.VMEM((N, D), jnp.float32),
            pltpu.VMEM((8, D), jnp.float32),
        ],
    )(agg, degin, x, W_gcn, gamma_bn.reshape(1, D), beta_bn.reshape(1, D),
      W_film, b_film.reshape(1, 2 * D))
    return out


# speedup vs baseline: 1.2509x; 1.1925x over previous
import functools

import jax
import jax.numpy as jnp
from jax import lax
from jax.experimental import pallas as pl
from jax.experimental.pallas import tpu as pltpu
from jax.experimental.pallas import tpu_sc as plsc

N = 10000
E = 320000
D = 128
DH = D // 2
NC = 2
NS = 16
NW = NC * NS
EPT = E // NW
CH = 80
NCHH = EPT // CH
GRP = 5
CA = 128
NFA = EPT // CA
TAIL = EPT - NFA * CA
NP = 10240
RPT = NP // NS
RC = 128
NRC = RPT // RC
BN_EPS = 1e-4

_mesh = plsc.VectorSubcoreMesh(core_axis_name="c", subcore_axis_name="s")
_sc_params = pltpu.CompilerParams(use_tc_tiling_on_sc=False,
                                  needs_layout_passes=False)


@functools.partial(
    pl.kernel,
    out_type=jax.ShapeDtypeStruct((NC, 2, NP, 16), jnp.float32),
    mesh=_mesh,
    compiler_params=_sc_params,
    scratch_types=[
        pltpu.VMEM((EPT,), jnp.int32),
        pltpu.VMEM((CH, 16), jnp.float32),
        pltpu.VMEM((RPT, 16), jnp.float32),
        pltpu.VMEM_SHARED((NP, 16), jnp.float32),
        pltpu.VMEM_SHARED((NP, 16), jnp.float32),
        pltpu.SemaphoreType.DMA,
    ],
)
def _sc_degrees(ei_hbm, deg_hbm, idx_v, ones_v, buf_v, degin_sh, degout_sh,
                sem):
    c = lax.axis_index("c")
    s = lax.axis_index("s")
    base = (c * NS + s) * EPT

    def fill(i, carry):
        buf_v[i, :] = jnp.zeros((16,), jnp.float32)
        return carry

    lax.fori_loop(0, RPT, fill, 0)

    def fill_ones(i, carry):
        ones_v[i, :] = jnp.ones((16,), jnp.float32)
        return carry

    lax.fori_loop(0, CH, fill_ones, 0)
    pltpu.sync_copy(buf_v, degin_sh.at[pl.ds(s * RPT, RPT)])
    pltpu.sync_copy(buf_v, degout_sh.at[pl.ds(s * RPT, RPT)])
    plsc.subcore_barrier()

    for row, deg_sh in ((1, degin_sh), (0, degout_sh)):
        pltpu.sync_copy(ei_hbm.at[row, pl.ds(base, EPT)], idx_v)

        def group(g, carry):
            for k in range(GRP):
                pltpu.async_copy(
                    ones_v, deg_sh.at[idx_v.at[pl.ds((g * GRP + k) * CH, CH)]],
                    sem, add=True)
            for k in range(GRP):
                pltpu.make_async_copy(ones_v,
                                      deg_sh.at[idx_v.at[pl.ds(0, CH)]],
                                      sem).wait()
            return carry

        lax.fori_loop(0, NCHH // GRP, group, 0)

    plsc.subcore_barrier()
    pltpu.sync_copy(degin_sh.at[pl.ds(s * RPT, RPT)], buf_v)
    pltpu.sync_copy(buf_v, deg_hbm.at[c, 0, pl.ds(s * RPT, RPT)])
    pltpu.sync_copy(degout_sh.at[pl.ds(s * RPT, RPT)], buf_v)
    pltpu.sync_copy(buf_v, deg_hbm.at[c, 1, pl.ds(s * RPT, RPT)])


@functools.partial(
    pl.kernel,
    out_type=jax.ShapeDtypeStruct((NC, NP, D), jnp.float32),
    mesh=_mesh,
    compiler_params=_sc_params,
    scratch_types=[
        pltpu.VMEM((EPT,), jnp.int32),
        pltpu.VMEM((EPT,), jnp.int32),
        pltpu.VMEM((CA, DH), jnp.float32),
        pltpu.VMEM((CA, DH), jnp.float32),
        pltpu.VMEM((RC, DH), jnp.float32),
        pltpu.VMEM_SHARED((NP, DH), jnp.float32),
        pltpu.SemaphoreType.DMA,
    ],
)
def _sc_aggregate(xs2_hbm, ei_hbm, agg_hbm, srcs_v, dsts_v, rows0_v, rows1_v,
                  buf_v, agg_sh, sem):
    c = lax.axis_index("c")
    s = lax.axis_index("s")
    base = (c * NS + s) * EPT
    pltpu.sync_copy(ei_hbm.at[0, pl.ds(base, EPT)], srcs_v)
    pltpu.sync_copy(ei_hbm.at[1, pl.ds(base, EPT)], dsts_v)

    def dbl(i, carry):
        srcs_v[pl.ds(i * 16, 16)] = srcs_v[pl.ds(i * 16, 16)] * 2
        return carry

    lax.fori_loop(0, EPT // 16, dbl, 0)

    def zb(i, carry):
        for j in range(DH // 16):
            buf_v[i, pl.ds(j * 16, 16)] = jnp.zeros((16,), jnp.float32)
        return carry

    def src_at(ch):
        return srcs_v.at[pl.ds(ch * CA, CA)]

    def dst_at(ch):
        return dsts_v.at[pl.ds(ch * CA, CA)]

    for half in (0, 1):
        if half == 1:
            def inc(i, carry):
                srcs_v[pl.ds(i * 16, 16)] = srcs_v[pl.ds(i * 16, 16)] + 1
                return carry

            lax.fori_loop(0, EPT // 16, inc, 0)

        lax.fori_loop(0, RC, zb, 0)
        for k in range(NRC):
            pltpu.sync_copy(buf_v, agg_sh.at[pl.ds(s * RPT + k * RC, RC)])
        plsc.subcore_barrier()

        pltpu.async_copy(xs2_hbm.at[src_at(0)], rows0_v, sem)

        def pair(h, carry):
            ch0 = 2 * h
            ch1 = ch0 + 1
            pltpu.async_copy(xs2_hbm.at[src_at(ch1)], rows1_v, sem)
            pltpu.make_async_copy(xs2_hbm.at[src_at(ch0)], rows0_v,
                                  sem).wait()
            pltpu.sync_copy(rows0_v, agg_sh.at[dst_at(ch0)], add=True)

            @pl.when(ch1 + 1 < NFA)
            def _():
                pltpu.async_copy(xs2_hbm.at[src_at(ch1 + 1)], rows0_v, sem)

            pltpu.make_async_copy(xs2_hbm.at[src_at(ch1)], rows1_v,
                                  sem).wait()
            pltpu.sync_copy(rows1_v, agg_sh.at[dst_at(ch1)], add=True)
            return carry

        lax.fori_loop(0, NFA // 2, pair, 0)
        tsrc = srcs_v.at[pl.ds(NFA * CA, TAIL)]
        tdst = dsts_v.at[pl.ds(NFA * CA, TAIL)]
        trows = rows0_v.at[pl.ds(0, TAIL)]
        pltpu.sync_copy(xs2_hbm.at[tsrc], trows)
        pltpu.sync_copy(trows, agg_sh.at[tdst], add=True)
        plsc.subcore_barrier()
        for k in range(NRC):
            r0 = s * RPT + k * RC
            pltpu.sync_copy(agg_sh.at[pl.ds(r0, RC)], buf_v)
            pltpu.sync_copy(buf_v,
                            agg_hbm.at[c, pl.ds(r0, RC),
                                       pl.ds(half * DH, DH)])
        plsc.subcore_barrier()


def _tc_prescale_body(x_ref, deg_ref, xs_ref):
    deg_out = deg_ref[0, 1, :N, 0:1] + deg_ref[1, 1, :N, 0:1]
    r_out = lax.rsqrt(jnp.maximum(deg_out, 1.0))
    xs_ref[...] = x_ref[...] * r_out


RB = 1000
NB = N // RB


def _tc_dense_body(agg_ref, deg_ref, x_ref, wg_ref, gb_ref, bb_ref, wf_ref,
                   bf_ref, o_ref, y_sc, sums_sc):
    p = pl.program_id(0)
    i = pl.program_id(1)

    @pl.when(p == 0)
    def _():
        deg_in = deg_ref[0, 0, :, 0:1] + deg_ref[1, 0, :, 0:1]
        r_in = lax.rsqrt(jnp.maximum(deg_in, 1.0))
        agg = (agg_ref[0] + agg_ref[1]) * r_in
        y = jnp.dot(agg, wg_ref[...], preferred_element_type=jnp.float32,
                    precision=lax.Precision.HIGHEST)
        y_sc[pl.ds(i * RB, RB), :] = y

        @pl.when(i == 0)
        def _():
            sums_sc[...] = jnp.zeros_like(sums_sc)

        sums_sc[0:1, :] += jnp.sum(y, axis=0, keepdims=True)
        sums_sc[1:2, :] += jnp.sum(y * y, axis=0, keepdims=True)

    @pl.when(p == 1)
    def _():
        y = y_sc[pl.ds(i * RB, RB), :]
        mean = sums_sc[0:1, :] * (1.0 / N)
        var = sums_sc[1:2, :] * (1.0 / N) - mean * mean
        yn = gb_ref[...] * (y - mean) * lax.rsqrt(var + BN_EPS) + bb_ref[...]
        film = jnp.dot(yn, wf_ref[...], preferred_element_type=jnp.float32,
                       precision=lax.Precision.HIGHEST) + bf_ref[...]
        z = film[:, :D] * yn + film[:, D:]
        o_ref[...] = jnp.maximum(z, 0.0) + x_ref[...]


def kernel(x, edge_index, W_gcn, gamma_bn, beta_bn, W_film, b_film):
    ei = edge_index.astype(jnp.int32)
    deg = _sc_degrees(ei)
    xs = pl.pallas_call(
        _tc_prescale_body,
        out_shape=jax.ShapeDtypeStruct((N, D), jnp.float32),
    )(x, deg)
    agg = _sc_aggregate(xs.reshape(2 * N, DH), ei)
    out = pl.pallas_call(
        _tc_dense_body,
        grid=(2, NB),
        in_specs=[
            pl.BlockSpec((NC, RB, D), lambda p, i: (0, i * (1 - p), 0)),
            pl.BlockSpec((NC, 2, RB, 16), lambda p, i: (0, 0, i * (1 - p), 0)),
            pl.BlockSpec((RB, D), lambda p, i: (i * p, 0)),
            pl.BlockSpec((D, D), lambda p, i: (0, 0)),
            pl.BlockSpec((1, D), lambda p, i: (0, 0)),
            pl.BlockSpec((1, D), lambda p, i: (0, 0)),
            pl.BlockSpec((D, 2 * D), lambda p, i: (0, 0)),
            pl.BlockSpec((1, 2 * D), lambda p, i: (0, 0)),
        ],
        out_specs=pl.BlockSpec((RB, D), lambda p, i: (i * p, 0)),
        out_shape=jax.ShapeDtypeStruct((N, D), jnp.float32),
        scratch_shapes=[
            pltpu.VMEM((N, D), jnp.float32),
            pltpu.VMEM((8, D), jnp.float32),
        ],
    )(agg, deg, x, W_gcn, gamma_bn.reshape(1, D), beta_bn.reshape(1, D),
      W_film, b_film.reshape(1, 2 * D))
    return out


# speedup vs baseline: 1.2936x; 1.0341x over previous
import functools

import jax
import jax.numpy as jnp
from jax import lax
from jax.experimental import pallas as pl
from jax.experimental.pallas import tpu as pltpu
from jax.experimental.pallas import tpu_sc as plsc

N = 10000
E = 320000
D = 128
DH = D // 2
NC = 2
NS = 16
NW = NC * NS
EPT = E // NW
CH = 128
NFH = EPT // CH
HTAIL = EPT - NFH * CH
GRP = 6
NGH = NFH // GRP
CA = 128
NFA = EPT // CA
TAIL = EPT - NFA * CA
NP = 10240
RPT = NP // NS
RC = 128
NRC = RPT // RC
BN_EPS = 1e-4

_mesh = plsc.VectorSubcoreMesh(core_axis_name="c", subcore_axis_name="s")
_sc_params = pltpu.CompilerParams(use_tc_tiling_on_sc=False,
                                  needs_layout_passes=False)


@functools.partial(
    pl.kernel,
    out_type=jax.ShapeDtypeStruct((NC, 2, NP, 16), jnp.float32),
    mesh=_mesh,
    compiler_params=_sc_params,
    scratch_types=[
        pltpu.VMEM((EPT,), jnp.int32),
        pltpu.VMEM((CH, 16), jnp.float32),
        pltpu.VMEM((RPT, 16), jnp.float32),
        pltpu.VMEM_SHARED((NP, 16), jnp.float32),
        pltpu.VMEM_SHARED((NP, 16), jnp.float32),
        pltpu.SemaphoreType.DMA,
    ],
)
def _sc_degrees(ei_hbm, deg_hbm, idx_v, ones_v, buf_v, degin_sh, degout_sh,
                sem):
    c = lax.axis_index("c")
    s = lax.axis_index("s")
    base = (c * NS + s) * EPT

    def fill(i, carry):
        buf_v[i, :] = jnp.zeros((16,), jnp.float32)
        return carry

    lax.fori_loop(0, RPT, fill, 0)

    def fill_ones(i, carry):
        ones_v[i, :] = jnp.ones((16,), jnp.float32)
        return carry

    lax.fori_loop(0, CH, fill_ones, 0)
    pltpu.sync_copy(buf_v, degin_sh.at[pl.ds(s * RPT, RPT)])
    pltpu.sync_copy(buf_v, degout_sh.at[pl.ds(s * RPT, RPT)])
    plsc.subcore_barrier()

    for row, deg_sh in ((1, degin_sh), (0, degout_sh)):
        pltpu.sync_copy(ei_hbm.at[row, pl.ds(base, EPT)], idx_v)

        def group(g, carry):
            for k in range(GRP):
                pltpu.async_copy(
                    ones_v, deg_sh.at[idx_v.at[pl.ds((g * GRP + k) * CH, CH)]],
                    sem, add=True)
            for k in range(GRP):
                pltpu.make_async_copy(ones_v,
                                      deg_sh.at[idx_v.at[pl.ds(0, CH)]],
                                      sem).wait()
            return carry

        lax.fori_loop(0, NGH, group, 0)
        pltpu.sync_copy(ones_v.at[pl.ds(0, HTAIL)],
                        deg_sh.at[idx_v.at[pl.ds(NFH * CH, HTAIL)]],
                        add=True)

    plsc.subcore_barrier()
    pltpu.sync_copy(degin_sh.at[pl.ds(s * RPT, RPT)], buf_v)
    pltpu.sync_copy(buf_v, deg_hbm.at[c, 0, pl.ds(s * RPT, RPT)])
    pltpu.sync_copy(degout_sh.at[pl.ds(s * RPT, RPT)], buf_v)
    pltpu.sync_copy(buf_v, deg_hbm.at[c, 1, pl.ds(s * RPT, RPT)])


@functools.partial(
    pl.kernel,
    out_type=jax.ShapeDtypeStruct((NC, NP, D), jnp.float32),
    mesh=_mesh,
    compiler_params=_sc_params,
    scratch_types=[
        pltpu.VMEM((EPT,), jnp.int32),
        pltpu.VMEM((EPT,), jnp.int32),
        pltpu.VMEM((CA, DH), jnp.float32),
        pltpu.VMEM((CA, DH), jnp.float32),
        pltpu.VMEM((RC, DH), jnp.float32),
        pltpu.VMEM_SHARED((NP, DH), jnp.float32),
        pltpu.SemaphoreType.DMA,
    ],
)
def _sc_aggregate(xs2_hbm, ei_hbm, agg_hbm, srcs_v, dsts_v, rows0_v, rows1_v,
                  buf_v, agg_sh, sem):
    c = lax.axis_index("c")
    s = lax.axis_index("s")
    base = (c * NS + s) * EPT
    pltpu.sync_copy(ei_hbm.at[0, pl.ds(base, EPT)], srcs_v)
    pltpu.sync_copy(ei_hbm.at[1, pl.ds(base, EPT)], dsts_v)

    def dbl(i, carry):
        srcs_v[pl.ds(i * 16, 16)] = srcs_v[pl.ds(i * 16, 16)] * 2
        return carry

    lax.fori_loop(0, EPT // 16, dbl, 0)

    def zb(i, carry):
        for j in range(DH // 16):
            buf_v[i, pl.ds(j * 16, 16)] = jnp.zeros((16,), jnp.float32)
        return carry

    def src_at(ch):
        return srcs_v.at[pl.ds(ch * CA, CA)]

    def dst_at(ch):
        return dsts_v.at[pl.ds(ch * CA, CA)]

    for half in (0, 1):
        if half == 1:
            def inc(i, carry):
                srcs_v[pl.ds(i * 16, 16)] = srcs_v[pl.ds(i * 16, 16)] + 1
                return carry

            lax.fori_loop(0, EPT // 16, inc, 0)

        lax.fori_loop(0, RC, zb, 0)
        for k in range(NRC):
            pltpu.sync_copy(buf_v, agg_sh.at[pl.ds(s * RPT + k * RC, RC)])
        plsc.subcore_barrier()

        pltpu.async_copy(xs2_hbm.at[src_at(0)], rows0_v, sem)

        def pair(h, carry):
            ch0 = 2 * h
            ch1 = ch0 + 1
            pltpu.async_copy(xs2_hbm.at[src_at(ch1)], rows1_v, sem)
            pltpu.make_async_copy(xs2_hbm.at[src_at(ch0)], rows0_v,
                                  sem).wait()
            pltpu.sync_copy(rows0_v, agg_sh.at[dst_at(ch0)], add=True)

            @pl.when(ch1 + 1 < NFA)
            def _():
                pltpu.async_copy(xs2_hbm.at[src_at(ch1 + 1)], rows0_v, sem)

            pltpu.make_async_copy(xs2_hbm.at[src_at(ch1)], rows1_v,
                                  sem).wait()
            pltpu.sync_copy(rows1_v, agg_sh.at[dst_at(ch1)], add=True)
            return carry

        lax.fori_loop(0, NFA // 2, pair, 0)
        tsrc = srcs_v.at[pl.ds(NFA * CA, TAIL)]
        tdst = dsts_v.at[pl.ds(NFA * CA, TAIL)]
        trows = rows0_v.at[pl.ds(0, TAIL)]
        pltpu.sync_copy(xs2_hbm.at[tsrc], trows)
        pltpu.sync_copy(trows, agg_sh.at[tdst], add=True)
        plsc.subcore_barrier()
        for k in range(NRC):
            r0 = s * RPT + k * RC
            pltpu.sync_copy(agg_sh.at[pl.ds(r0, RC)], buf_v)
            pltpu.sync_copy(buf_v,
                            agg_hbm.at[c, pl.ds(r0, RC),
                                       pl.ds(half * DH, DH)])
        plsc.subcore_barrier()


def _tc_prescale_body(x_ref, deg_ref, xs_ref):
    deg_out = deg_ref[0, 1, :N, 0:1] + deg_ref[1, 1, :N, 0:1]
    r_out = lax.rsqrt(jnp.maximum(deg_out, 1.0))
    xs_ref[...] = x_ref[...] * r_out


RB = 1000
NB = N // RB


def _tc_dense_body(agg_ref, deg_ref, x_ref, wg_ref, gb_ref, bb_ref, wf_ref,
                   bf_ref, o_ref, y_sc, sums_sc):
    p = pl.program_id(0)
    i = pl.program_id(1)

    @pl.when(p == 0)
    def _():
        deg_in = deg_ref[0, 0, :, 0:1] + deg_ref[1, 0, :, 0:1]
        r_in = lax.rsqrt(jnp.maximum(deg_in, 1.0))
        agg = (agg_ref[0] + agg_ref[1]) * r_in
        y = jnp.dot(agg, wg_ref[...], preferred_element_type=jnp.float32)
        y_sc[pl.ds(i * RB, RB), :] = y

        @pl.when(i == 0)
        def _():
            sums_sc[...] = jnp.zeros_like(sums_sc)

        sums_sc[0:1, :] += jnp.sum(y, axis=0, keepdims=True)
        sums_sc[1:2, :] += jnp.sum(y * y, axis=0, keepdims=True)

    @pl.when(p == 1)
    def _():
        y = y_sc[pl.ds(i * RB, RB), :]
        mean = sums_sc[0:1, :] * (1.0 / N)
        var = sums_sc[1:2, :] * (1.0 / N) - mean * mean
        yn = gb_ref[...] * (y - mean) * lax.rsqrt(var + BN_EPS) + bb_ref[...]
        film = jnp.dot(yn, wf_ref[...],
                       preferred_element_type=jnp.float32) + bf_ref[...]
        z = film[:, :D] * yn + film[:, D:]
        o_ref[...] = jnp.maximum(z, 0.0) + x_ref[...]


def kernel(x, edge_index, W_gcn, gamma_bn, beta_bn, W_film, b_film):
    ei = edge_index.astype(jnp.int32)
    deg = _sc_degrees(ei)
    xs = pl.pallas_call(
        _tc_prescale_body,
        out_shape=jax.ShapeDtypeStruct((N, D), jnp.float32),
    )(x, deg)
    agg = _sc_aggregate(xs.reshape(2 * N, DH), ei)
    out = pl.pallas_call(
        _tc_dense_body,
        grid=(2, NB),
        in_specs=[
            pl.BlockSpec((NC, RB, D), lambda p, i: (0, i * (1 - p), 0)),
            pl.BlockSpec((NC, 2, RB, 16), lambda p, i: (0, 0, i * (1 - p), 0)),
            pl.BlockSpec((RB, D), lambda p, i: (i * p, 0)),
            pl.BlockSpec((D, D), lambda p, i: (0, 0)),
            pl.BlockSpec((1, D), lambda p, i: (0, 0)),
            pl.BlockSpec((1, D), lambda p, i: (0, 0)),
            pl.BlockSpec((D, 2 * D), lambda p, i: (0, 0)),
            pl.BlockSpec((1, 2 * D), lambda p, i: (0, 0)),
        ],
        out_specs=pl.BlockSpec((RB, D), lambda p, i: (i * p, 0)),
        out_shape=jax.ShapeDtypeStruct((N, D), jnp.float32),
        scratch_shapes=[
            pltpu.VMEM((N, D), jnp.float32),
            pltpu.VMEM((8, D), jnp.float32),
        ],
    )(agg, deg, x, W_gcn, gamma_bn.reshape(1, D), beta_bn.reshape(1, D),
      W_film, b_film.reshape(1, 2 * D))
    return out
